# baseline (speedup 1.0000x reference)
import math

import jax
import jax.numpy as jnp
from jax import lax
from jax.experimental import pallas as pl
from jax.experimental.pallas import tpu as pltpu
from jax.experimental.pallas import tpu_sc as plsc

_NC = 2
_NS = 16
_NW = _NC * _NS
_L = 16


def _sc_pe_add(B, S, D):
    PW = S // _NW
    scale = math.sqrt(D)
    mesh = plsc.VectorSubcoreMesh(core_axis_name="c", subcore_axis_name="s")

    def body(x_hbm, keep_hbm, pe_hbm, out_hbm,
             pe_v, keep_v, xin0, xin1, xout0, xout1,
             in_sem0, in_sem1, out_sem0, out_sem1):
        wid = lax.axis_index("s") * _NC + lax.axis_index("c")
        s0 = wid * PW
        pltpu.sync_copy(pe_hbm.at[pl.ds(s0, PW)], pe_v)
        pltpu.sync_copy(keep_hbm.at[wid], keep_v)

        xins = (xin0, xin1)
        xouts = (xout0, xout1)
        in_sems = (in_sem0, in_sem1)
        out_sems = (out_sem0, out_sem1)

        def in_copy(b, buf):
            return pltpu.make_async_copy(
                x_hbm.at[b, pl.ds(s0, PW)], xins[buf], in_sems[buf])

        def out_copy(b, buf):
            return pltpu.make_async_copy(
                xouts[buf], out_hbm.at[b, pl.ds(s0, PW)], out_sems[buf])

        in_copy(0, 0).start()
        in_copy(1, 1).start()

        def step(i, _):
            for buf in range(2):
                b = 2 * i + buf
                in_copy(b, buf).wait()

                @pl.when(b >= 2)
                def _wait_prev_out():
                    out_copy(b - 2, buf).wait()

                kvec = keep_v[pl.ds(b * _L, _L)]
                for r in range(PW):
                    kv = kvec[r]
                    for c in range(D // _L):
                        sl = pl.ds(c * _L, _L)
                        xv = xins[buf][r, sl]
                        pv = pe_v[r, sl]
                        xouts[buf][r, sl] = scale * xv + kv * pv

                out_copy(b, buf).start()

                @pl.when(b + 2 < B)
                def _next_in():
                    in_copy(b + 2, buf).start()
            return 0

        lax.fori_loop(0, B // 2, step, 0)
        out_copy(B - 2, 0).wait()
        out_copy(B - 1, 1).wait()

    return pl.kernel(
        body,
        out_type=jax.ShapeDtypeStruct((B, S, D), jnp.float32),
        mesh=mesh,
        scratch_types=[
            pltpu.VMEM((PW, D), jnp.float32),
            pltpu.VMEM((B * _L,), jnp.float32),
            pltpu.VMEM((PW, D), jnp.float32),
            pltpu.VMEM((PW, D), jnp.float32),
            pltpu.VMEM((PW, D), jnp.float32),
            pltpu.VMEM((PW, D), jnp.float32),
            pltpu.SemaphoreType.DMA,
            pltpu.SemaphoreType.DMA,
            pltpu.SemaphoreType.DMA,
            pltpu.SemaphoreType.DMA,
        ],
    )


def kernel(x, mask, pe):
    B, S, D = x.shape
    keep = (mask != 1).astype(jnp.float32)
    PW = S // _NW
    keep_w = keep.reshape(B, _NW, PW).transpose(1, 0, 2)
    keep_packed = jnp.pad(keep_w, ((0, 0), (0, 0), (0, _L - PW)))
    keep_packed = keep_packed.reshape(_NW, B * _L)
    pe_s = pe[:S]
    return _sc_pe_add(B, S, D)(x, keep_packed, pe_s)


# speedup vs baseline: 2.7361x; 2.7361x over previous
import math

import jax
import jax.numpy as jnp
from jax import lax
from jax.experimental import pallas as pl
from jax.experimental.pallas import tpu as pltpu

_CHUNK = 1024
_NBUF = 4


def _pe_add_manual(x_hbm, keep_hbm, pe_hbm, out_hbm,
                   pe_v, keep_v, xin, xout, in_sems, out_sems):
    R, D = x_hbm.shape
    S = pe_v.shape[0]
    NCHUNKS = R // _CHUNK
    scale = math.sqrt(D)

    pltpu.make_async_copy(pe_hbm, pe_v, in_sems.at[0]).start()
    pltpu.make_async_copy(keep_hbm, keep_v, in_sems.at[1]).start()
    pltpu.make_async_copy(pe_hbm, pe_v, in_sems.at[0]).wait()
    pltpu.make_async_copy(keep_hbm, keep_v, in_sems.at[1]).wait()

    def in_copy(c, k):
        return pltpu.make_async_copy(
            x_hbm.at[pl.ds(c * _CHUNK, _CHUNK), :], xin.at[k], in_sems.at[k])

    def out_copy(c, k):
        return pltpu.make_async_copy(
            xout.at[k], out_hbm.at[pl.ds(c * _CHUNK, _CHUNK), :], out_sems.at[k])

    for k in range(_NBUF):
        in_copy(k, k).start()

    pe_rows = pe_v[...]

    def step(m, _):
        for k in range(_NBUF):
            c = m * _NBUF + k
            in_copy(c, k).wait()

            @pl.when(c >= _NBUF)
            def _wait_prev_out():
                out_copy(c - _NBUF, k).wait()

            for j in range(_CHUNK // S):
                sl = pl.ds(j * S, S)
                keep = keep_v[pl.ds(c * _CHUNK + j * S, S), :]
                xout[k, sl, :] = scale * xin[k, sl, :] + keep * pe_rows

            out_copy(c, k).start()

            @pl.when(c + _NBUF < NCHUNKS)
            def _next_in():
                in_copy(c + _NBUF, k).start()
        return 0

    lax.fori_loop(0, NCHUNKS // _NBUF, step, 0)
    for k in range(_NBUF):
        out_copy(NCHUNKS - _NBUF + k, k).wait()


def kernel(x, mask, pe):
    B, S, D = x.shape
    x2 = x.reshape(B * S, D)
    keep = (mask != 1).astype(jnp.float32).reshape(B * S, 1)
    pe_s = pe[:S]

    out = pl.pallas_call(
        _pe_add_manual,
        in_specs=[
            pl.BlockSpec(memory_space=pl.ANY),
            pl.BlockSpec(memory_space=pl.ANY),
            pl.BlockSpec(memory_space=pl.ANY),
        ],
        out_specs=pl.BlockSpec(memory_space=pl.ANY),
        out_shape=jax.ShapeDtypeStruct((B * S, D), x.dtype),
        scratch_shapes=[
            pltpu.VMEM((S, D), jnp.float32),
            pltpu.VMEM((B * S, 1), jnp.float32),
            pltpu.VMEM((_NBUF, _CHUNK, D), jnp.float32),
            pltpu.VMEM((_NBUF, _CHUNK, D), jnp.float32),
            pltpu.SemaphoreType.DMA((_NBUF,)),
            pltpu.SemaphoreType.DMA((_NBUF,)),
        ],
    )(x2, keep, pe_s)
    return out.reshape(B, S, D)


# speedup vs baseline: 2.7707x; 1.0127x over previous
import math

import jax
import jax.numpy as jnp
from jax import lax
from jax.experimental import pallas as pl
from jax.experimental.pallas import tpu as pltpu

_CHUNK = 1024
_NBUF = 4


def _pe_add_manual(x_hbm, mask_hbm, pe_hbm, out_hbm,
                   pe_v, mask_v, xin, xout, in_sems, out_sems):
    R, D = x_hbm.shape
    S = pe_v.shape[0]
    NCHUNKS = R // _CHUNK
    scale = math.sqrt(D)

    pe_copy = pltpu.make_async_copy(
        pe_hbm.at[pl.ds(0, S), :], pe_v, in_sems.at[0])
    mask_copy = pltpu.make_async_copy(mask_hbm, mask_v, in_sems.at[1])
    pe_copy.start()
    mask_copy.start()
    pe_copy.wait()
    mask_copy.wait()

    def in_copy(c, k):
        return pltpu.make_async_copy(
            x_hbm.at[pl.ds(c * _CHUNK, _CHUNK), :], xin.at[k], in_sems.at[k])

    def out_copy(c, k):
        return pltpu.make_async_copy(
            xout.at[k], out_hbm.at[pl.ds(c * _CHUNK, _CHUNK), :], out_sems.at[k])

    for k in range(_NBUF):
        in_copy(k, k).start()

    pe_rows = pe_v[...]

    def step(m, _):
        for k in range(_NBUF):
            c = m * _NBUF + k
            in_copy(c, k).wait()

            @pl.when(c >= _NBUF)
            def _wait_prev_out():
                out_copy(c - _NBUF, k).wait()

            for j in range(_CHUNK // S):
                sl = pl.ds(j * S, S)
                mrows = mask_v[pl.ds(c * _CHUNK + j * S, S), :]
                keep = (mrows != 1).astype(jnp.float32)
                xout[k, sl, :] = scale * xin[k, sl, :] + keep * pe_rows

            out_copy(c, k).start()

            @pl.when(c + _NBUF < NCHUNKS)
            def _next_in():
                in_copy(c + _NBUF, k).start()
        return 0

    lax.fori_loop(0, NCHUNKS // _NBUF, step, 0)
    for k in range(_NBUF):
        out_copy(NCHUNKS - _NBUF + k, k).wait()


def kernel(x, mask, pe):
    B, S, D = x.shape
    x2 = x.reshape(B * S, D)
    mask2 = mask.reshape(B * S, 1)

    out = pl.pallas_call(
        _pe_add_manual,
        in_specs=[
            pl.BlockSpec(memory_space=pl.ANY),
            pl.BlockSpec(memory_space=pl.ANY),
            pl.BlockSpec(memory_space=pl.ANY),
        ],
        out_specs=pl.BlockSpec(memory_space=pl.ANY),
        out_shape=jax.ShapeDtypeStruct((B * S, D), x.dtype),
        scratch_shapes=[
            pltpu.VMEM((S, D), jnp.float32),
            pltpu.VMEM((B * S, 1), jnp.int32),
            pltpu.VMEM((_NBUF, _CHUNK, D), jnp.float32),
            pltpu.VMEM((_NBUF, _CHUNK, D), jnp.float32),
            pltpu.SemaphoreType.DMA((_NBUF,)),
            pltpu.SemaphoreType.DMA((_NBUF,)),
        ],
    )(x2, mask2, pe)
    return out.reshape(B, S, D)


# speedup vs baseline: 2.7868x; 1.0058x over previous
import math

import jax
import jax.numpy as jnp
from jax import lax
from jax.experimental import pallas as pl
from jax.experimental.pallas import tpu as pltpu

_CHUNK = 1024
_NBUF = 4


def _pe_add_manual(x_hbm, mask_hbm, pe_hbm, out_hbm,
                   pe_v, mask_v, xin, xout, in_sems, out_sems):
    R, D = x_hbm.shape
    S = pe_v.shape[0]
    NCHUNKS = R // _CHUNK
    scale = math.sqrt(D)

    def in_copy(c, k):
        return pltpu.make_async_copy(
            x_hbm.at[pl.ds(c * _CHUNK, _CHUNK), :], xin.at[k], in_sems.at[k])

    def out_copy(c, k):
        return pltpu.make_async_copy(
            xout.at[k], out_hbm.at[pl.ds(c * _CHUNK, _CHUNK), :], out_sems.at[k])

    pe_copy = pltpu.make_async_copy(
        pe_hbm.at[pl.ds(0, S), :], pe_v, out_sems.at[0])
    mask_copy = pltpu.make_async_copy(mask_hbm, mask_v, out_sems.at[1])
    pe_copy.start()
    mask_copy.start()
    for k in range(_NBUF):
        in_copy(k, k).start()
    pe_copy.wait()
    mask_copy.wait()

    pe_rows = pe_v[...]

    def step(m, _):
        for k in range(_NBUF):
            c = m * _NBUF + k
            in_copy(c, k).wait()

            @pl.when(c >= _NBUF)
            def _wait_prev_out():
                out_copy(c - _NBUF, k).wait()

            for j in range(_CHUNK // S):
                sl = pl.ds(j * S, S)
                mrows = mask_v[pl.ds(c * _CHUNK + j * S, S), :]
                keep = (mrows != 1).astype(jnp.float32)
                xout[k, sl, :] = scale * xin[k, sl, :] + keep * pe_rows

            out_copy(c, k).start()

            @pl.when(c + _NBUF < NCHUNKS)
            def _next_in():
                in_copy(c + _NBUF, k).start()
        return 0

    lax.fori_loop(0, NCHUNKS // _NBUF, step, 0)
    for k in range(_NBUF):
        out_copy(NCHUNKS - _NBUF + k, k).wait()


def kernel(x, mask, pe):
    B, S, D = x.shape
    x2 = x.reshape(B * S, D)
    mask2 = mask.reshape(B * S, 1)

    out = pl.pallas_call(
        _pe_add_manual,
        in_specs=[
            pl.BlockSpec(memory_space=pl.ANY),
            pl.BlockSpec(memory_space=pl.ANY),
            pl.BlockSpec(memory_space=pl.ANY),
        ],
        out_specs=pl.BlockSpec(memory_space=pl.ANY),
        out_shape=jax.ShapeDtypeStruct((B * S, D), x.dtype),
        scratch_shapes=[
            pltpu.VMEM((S, D), jnp.float32),
            pltpu.VMEM((B * S, 1), jnp.int32),
            pltpu.VMEM((_NBUF, _CHUNK, D), jnp.float32),
            pltpu.VMEM((_NBUF, _CHUNK, D), jnp.float32),
            pltpu.SemaphoreType.DMA((_NBUF,)),
            pltpu.SemaphoreType.DMA((_NBUF,)),
        ],
    )(x2, mask2, pe)
    return out.reshape(B, S, D)
